# baseline (speedup 1.0000x reference)
import functools

import jax
import jax.numpy as jnp
from jax import lax
from jax.experimental import pallas as pl
from jax.experimental.pallas import tpu as pltpu
from jax.experimental.pallas import tpu_sc as plsc

B = 4096
NL = 10
CH = 6400
LANES = 16
NW = 32
ROWS_PER_W = B // NW
CHUNKS = CH // LANES
GR = 4


def _sc_partials_kernel(z_hbm, labels_hbm, a_out, q_out, c_out,
                        a_v, zbuf0, zbuf1, labels_v, q_v, c_v, sem0, sem1):
    nc = 2
    wid = lax.axis_index("s") * nc + lax.axis_index("c")
    base = wid * ROWS_PER_W

    zeros = jnp.zeros((LANES,), jnp.float32)
    ones = jnp.ones((LANES,), jnp.float32)

    pltpu.sync_copy(labels_hbm.at[pl.ds(base, ROWS_PER_W)], labels_v)

    def zero_body(i, c):
        a_v[pl.ds(LANES * i, LANES)] = zeros
        return c
    lax.fori_loop(0, NL * CHUNKS, zero_body, 0)
    for l in range(NL):
        q_v[pl.ds(LANES * l, LANES)] = zeros
        c_v[pl.ds(LANES * l, LANES)] = zeros

    bufs = (zbuf0, zbuf1)
    sems = (sem0, sem1)

    def start(k, sb):
        pltpu.make_async_copy(z_hbm.at[pl.ds((base + GR * sb) * CH, GR * CH)],
                              bufs[k], sems[k]).start()

    def wait(k):
        pltpu.make_async_copy(z_hbm.at[pl.ds(base * CH, GR * CH)], bufs[k],
                              sems[k]).wait()

    def process(buf, ubase, lab):
        off = lab * CH

        def body(j, qs):
            q0, q1, q2, q3 = qs
            o = 4 * LANES * j
            zv0 = buf[pl.ds(ubase + o, LANES)]
            plsc.addupdate(a_v.at[pl.ds(off + o, LANES)], zv0)
            zv1 = buf[pl.ds(ubase + o + LANES, LANES)]
            plsc.addupdate(a_v.at[pl.ds(off + o + LANES, LANES)], zv1)
            zv2 = buf[pl.ds(ubase + o + 2 * LANES, LANES)]
            plsc.addupdate(a_v.at[pl.ds(off + o + 2 * LANES, LANES)], zv2)
            zv3 = buf[pl.ds(ubase + o + 3 * LANES, LANES)]
            plsc.addupdate(a_v.at[pl.ds(off + o + 3 * LANES, LANES)], zv3)
            return (q0 + zv0 * zv0, q1 + zv1 * zv1,
                    q2 + zv2 * zv2, q3 + zv3 * zv3)
        q0, q1, q2, q3 = lax.fori_loop(0, CHUNKS // 4, body, (zeros,) * 4)
        q = (q0 + q1) + (q2 + q3)
        plsc.addupdate(q_v.at[pl.ds(lab * LANES, LANES)], q)
        plsc.addupdate(c_v.at[pl.ds(lab * LANES, LANES)], ones)

    NGROUPS = ROWS_PER_W // LANES
    SB_PER_G = LANES // GR
    start(0, 0)
    start(1, 1)

    def do_group(g, last):
        lv = labels_v[pl.ds(LANES * g, LANES)]
        for t in range(SB_PER_G):
            k = t % 2
            wait(k)
            for u in range(GR):
                process(bufs[k], u * CH, lv[GR * t + u])
            if (not last) or t < 2:
                start(k, SB_PER_G * g + t + 2)

    def group_body(g, c):
        do_group(g, False)
        return c
    lax.fori_loop(0, NGROUPS - 1, group_body, 0)
    do_group(NGROUPS - 1, True)

    pltpu.sync_copy(a_v, a_out.at[wid])
    pltpu.sync_copy(q_v, q_out.at[wid])
    pltpu.sync_copy(c_v, c_out.at[wid])


def _finalize_body(a_ref, q_ref, c_ref, out_ref):
    a = jnp.sum(a_ref[...], axis=0)
    q = jnp.sum(q_ref[...], axis=(0, 2))
    n = jnp.sum(c_ref[...][:, :, 0], axis=0)
    ssq = jnp.sum(a * a, axis=1)
    safe = jnp.maximum(n, 1.0)
    chf = jnp.float32(CH)
    eps2 = jnp.float32(1e-16)
    mse = q / (safe * chf) - ssq / (safe * safe * chf) + eps2
    out_ref[...] = jnp.sum(jnp.where(n > 0, mse, 0.0)).reshape(1, 1)


@jax.jit
def _run(z2d, labels):
    mesh = plsc.VectorSubcoreMesh(core_axis_name="c", subcore_axis_name="s")
    sc = pl.kernel(
        _sc_partials_kernel,
        mesh=mesh,
        out_type=(
            jax.ShapeDtypeStruct((NW, NL * CH), jnp.float32),
            jax.ShapeDtypeStruct((NW, NL * LANES), jnp.float32),
            jax.ShapeDtypeStruct((NW, NL * LANES), jnp.float32),
        ),
        scratch_types=[
            pltpu.VMEM((NL * CH,), jnp.float32),
            pltpu.VMEM((GR * CH,), jnp.float32),
            pltpu.VMEM((GR * CH,), jnp.float32),
            pltpu.VMEM((ROWS_PER_W,), jnp.int32),
            pltpu.VMEM((NL * LANES,), jnp.float32),
            pltpu.VMEM((NL * LANES,), jnp.float32),
            pltpu.SemaphoreType.DMA,
            pltpu.SemaphoreType.DMA,
        ],
    )
    a_part, q_part, c_part = sc(z2d, labels)

    out = pl.pallas_call(
        _finalize_body,
        out_shape=jax.ShapeDtypeStruct((1, 1), jnp.float32),
    )(a_part.reshape(NW, NL, CH),
      q_part.reshape(NW, NL, LANES),
      c_part.reshape(NW, NL, LANES))
    return out[0, 0]


def kernel(z, labels):
    return _run(z.reshape(B * CH), labels)


# speedup vs baseline: 1.9915x; 1.9915x over previous
import functools

import jax
import jax.numpy as jnp
from jax import lax
from jax.experimental import pallas as pl
from jax.experimental.pallas import tpu as pltpu
from jax.experimental.pallas import tpu_sc as plsc

B = 4096
NL = 10
CH = 6400
LANES = 16
NW = 32
ROWS_PER_W = B // NW
CHUNKS = CH // LANES
GR = 4


def _sc_partials_kernel(z_hbm, labels_hbm, a_out, q_out, c_out,
                        a_v, zbuf0, zbuf1, labels_v, q_v, c_v, sem0, sem1):
    nc = 2
    wid = lax.axis_index("s") * nc + lax.axis_index("c")
    base = wid * ROWS_PER_W

    zeros = jnp.zeros((LANES,), jnp.float32)
    ones = jnp.ones((LANES,), jnp.float32)

    pltpu.sync_copy(labels_hbm.at[pl.ds(base, ROWS_PER_W)], labels_v)

    def zero_body(i, c):
        a_v[pl.ds(LANES * i, LANES)] = zeros
        return c
    lax.fori_loop(0, NL * CHUNKS, zero_body, 0)
    for l in range(NL):
        q_v[pl.ds(LANES * l, LANES)] = zeros
        c_v[pl.ds(LANES * l, LANES)] = zeros

    bufs = (zbuf0, zbuf1)
    sems = (sem0, sem1)

    def start(k, row):
        pltpu.make_async_copy(z_hbm.at[base + row], bufs[k], sems[k]).start()

    def wait(k):
        pltpu.make_async_copy(z_hbm.at[base], bufs[k], sems[k]).wait()

    UNR = 8

    def process(buf, lab):
        off = lab * CH

        def body(j, qs):
            q0, q1, q2, q3 = qs
            o = UNR * LANES * j
            zv = [buf[pl.ds(o + LANES * u, LANES)] for u in range(UNR)]
            for u in range(UNR):
                plsc.addupdate(a_v.at[pl.ds(off + o + LANES * u, LANES)],
                               zv[u])
            q0 = q0 + zv[0] * zv[0] + zv[4] * zv[4]
            q1 = q1 + zv[1] * zv[1] + zv[5] * zv[5]
            q2 = q2 + zv[2] * zv[2] + zv[6] * zv[6]
            q3 = q3 + zv[3] * zv[3] + zv[7] * zv[7]
            return (q0, q1, q2, q3)
        q0, q1, q2, q3 = lax.fori_loop(0, CHUNKS // UNR, body, (zeros,) * 4)
        q = (q0 + q1) + (q2 + q3)
        plsc.addupdate(q_v.at[pl.ds(lab * LANES, LANES)], q)
        plsc.addupdate(c_v.at[pl.ds(lab * LANES, LANES)], ones)

    NGROUPS = ROWS_PER_W // LANES
    start(0, 0)
    start(1, 1)

    def group_body(g, c):
        lv = labels_v[pl.ds(LANES * g, LANES)]
        for u in range(LANES):
            k = u % 2
            wait(k)
            process(bufs[k], lv[u])
            start(k, LANES * g + u + 2)
        return c
    lax.fori_loop(0, NGROUPS - 1, group_body, 0)
    lv = labels_v[pl.ds(LANES * (NGROUPS - 1), LANES)]
    for u in range(LANES):
        k = u % 2
        wait(k)
        process(bufs[k], lv[u])
        if u < LANES - 2:
            start(k, LANES * (NGROUPS - 1) + u + 2)

    pltpu.sync_copy(a_v, a_out.at[wid])
    pltpu.sync_copy(q_v, q_out.at[wid])
    pltpu.sync_copy(c_v, c_out.at[wid])


def _finalize_body(a_ref, q_ref, c_ref, out_ref):
    a = jnp.sum(a_ref[...], axis=0)
    q = jnp.sum(q_ref[...], axis=(0, 2))
    n = jnp.sum(c_ref[...][:, :, 0], axis=0)
    ssq = jnp.sum(a * a, axis=1)
    safe = jnp.maximum(n, 1.0)
    chf = jnp.float32(CH)
    eps2 = jnp.float32(1e-16)
    mse = q / (safe * chf) - ssq / (safe * safe * chf) + eps2
    out_ref[...] = jnp.sum(jnp.where(n > 0, mse, 0.0)).reshape(1, 1)


@jax.jit
def _run(z2d, labels):
    mesh = plsc.VectorSubcoreMesh(core_axis_name="c", subcore_axis_name="s")
    sc = pl.kernel(
        _sc_partials_kernel,
        mesh=mesh,
        out_type=(
            jax.ShapeDtypeStruct((NW, NL * CH), jnp.float32),
            jax.ShapeDtypeStruct((NW, NL * LANES), jnp.float32),
            jax.ShapeDtypeStruct((NW, NL * LANES), jnp.float32),
        ),
        scratch_types=[
            pltpu.VMEM((NL * CH,), jnp.float32),
            pltpu.VMEM((CH,), jnp.float32),
            pltpu.VMEM((CH,), jnp.float32),
            pltpu.VMEM((ROWS_PER_W,), jnp.int32),
            pltpu.VMEM((NL * LANES,), jnp.float32),
            pltpu.VMEM((NL * LANES,), jnp.float32),
            pltpu.SemaphoreType.DMA,
            pltpu.SemaphoreType.DMA,
        ],
    )
    a_part, q_part, c_part = sc(z2d, labels)

    out = pl.pallas_call(
        _finalize_body,
        out_shape=jax.ShapeDtypeStruct((1, 1), jnp.float32),
    )(a_part.reshape(NW, NL, CH),
      q_part.reshape(NW, NL, LANES),
      c_part.reshape(NW, NL, LANES))
    return out[0, 0]


def kernel(z, labels):
    return _run(z.reshape(B, CH), labels)


# speedup vs baseline: 2.1281x; 1.0686x over previous
import functools

import jax
import jax.numpy as jnp
from jax import lax
from jax.experimental import pallas as pl
from jax.experimental.pallas import tpu as pltpu
from jax.experimental.pallas import tpu_sc as plsc

B = 4096
NL = 10
CH = 6400
LANES = 16
NW = 32
ROWS_PER_W = B // NW
CHUNKS = CH // LANES
GR = 4


def _sc_partials_kernel(z_hbm, labels_hbm, a_out, q_out, c_out,
                        a_v, zbuf0, zbuf1, labels_v, q_v, c_v, sem0, sem1):
    nc = 2
    wid = lax.axis_index("s") * nc + lax.axis_index("c")
    base = wid * ROWS_PER_W

    zeros = jnp.zeros((LANES,), jnp.float32)
    ones = jnp.ones((LANES,), jnp.float32)

    pltpu.sync_copy(labels_hbm.at[pl.ds(base, ROWS_PER_W)], labels_v)

    def zero_body(i, c):
        for u in range(8):
            a_v[pl.ds(8 * LANES * i + LANES * u, LANES)] = zeros
        return c
    lax.fori_loop(0, NL * CHUNKS // 8, zero_body, 0)
    for l in range(NL):
        q_v[pl.ds(LANES * l, LANES)] = zeros
        c_v[pl.ds(LANES * l, LANES)] = zeros

    bufs = (zbuf0, zbuf1)
    sems = (sem0, sem1)

    def start(k, row):
        pltpu.make_async_copy(z_hbm.at[base + row], bufs[k], sems[k]).start()

    def wait(k):
        pltpu.make_async_copy(z_hbm.at[base], bufs[k], sems[k]).wait()

    UNR = 16
    NACC = 8

    def process(buf, lab):
        off = lab * CH

        def body(j, qs):
            o = UNR * LANES * j
            zv = [buf[pl.ds(o + LANES * u, LANES)] for u in range(UNR)]
            for u in range(UNR):
                plsc.addupdate(a_v.at[pl.ds(off + o + LANES * u, LANES)],
                               zv[u])
            qs = list(qs)
            for u in range(UNR):
                qs[u % NACC] = qs[u % NACC] + zv[u] * zv[u]
            return tuple(qs)
        qs = lax.fori_loop(0, CHUNKS // UNR, body, (zeros,) * NACC)
        q = ((qs[0] + qs[1]) + (qs[2] + qs[3])) + \
            ((qs[4] + qs[5]) + (qs[6] + qs[7]))
        plsc.addupdate(q_v.at[pl.ds(lab * LANES, LANES)], q)
        plsc.addupdate(c_v.at[pl.ds(lab * LANES, LANES)], ones)

    NGROUPS = ROWS_PER_W // LANES
    start(0, 0)
    start(1, 1)

    def group_body(g, c):
        lv = labels_v[pl.ds(LANES * g, LANES)]
        for u in range(LANES):
            k = u % 2
            wait(k)
            process(bufs[k], lv[u])
            start(k, LANES * g + u + 2)
        return c
    lax.fori_loop(0, NGROUPS - 1, group_body, 0)
    lv = labels_v[pl.ds(LANES * (NGROUPS - 1), LANES)]
    for u in range(LANES):
        k = u % 2
        wait(k)
        process(bufs[k], lv[u])
        if u < LANES - 2:
            start(k, LANES * (NGROUPS - 1) + u + 2)

    pltpu.sync_copy(a_v, a_out.at[wid])
    pltpu.sync_copy(q_v, q_out.at[wid])
    pltpu.sync_copy(c_v, c_out.at[wid])


def _finalize_body(a_ref, q_ref, c_ref, out_ref):
    a = jnp.sum(a_ref[...], axis=0)
    q = jnp.sum(q_ref[...], axis=(0, 2))
    n = jnp.sum(c_ref[...][:, :, 0], axis=0)
    ssq = jnp.sum(a * a, axis=1)
    safe = jnp.maximum(n, 1.0)
    chf = jnp.float32(CH)
    eps2 = jnp.float32(1e-16)
    mse = q / (safe * chf) - ssq / (safe * safe * chf) + eps2
    out_ref[...] = jnp.sum(jnp.where(n > 0, mse, 0.0)).reshape(1, 1)


@jax.jit
def _run(z2d, labels):
    mesh = plsc.VectorSubcoreMesh(core_axis_name="c", subcore_axis_name="s")
    sc = pl.kernel(
        _sc_partials_kernel,
        mesh=mesh,
        out_type=(
            jax.ShapeDtypeStruct((NW, NL * CH), jnp.float32),
            jax.ShapeDtypeStruct((NW, NL * LANES), jnp.float32),
            jax.ShapeDtypeStruct((NW, NL * LANES), jnp.float32),
        ),
        scratch_types=[
            pltpu.VMEM((NL * CH,), jnp.float32),
            pltpu.VMEM((CH,), jnp.float32),
            pltpu.VMEM((CH,), jnp.float32),
            pltpu.VMEM((ROWS_PER_W,), jnp.int32),
            pltpu.VMEM((NL * LANES,), jnp.float32),
            pltpu.VMEM((NL * LANES,), jnp.float32),
            pltpu.SemaphoreType.DMA,
            pltpu.SemaphoreType.DMA,
        ],
    )
    a_part, q_part, c_part = sc(z2d, labels)

    out = pl.pallas_call(
        _finalize_body,
        out_shape=jax.ShapeDtypeStruct((1, 1), jnp.float32),
    )(a_part.reshape(NW, NL, CH),
      q_part.reshape(NW, NL, LANES),
      c_part.reshape(NW, NL, LANES))
    return out[0, 0]


def kernel(z, labels):
    return _run(z.reshape(B, CH), labels)
